# Initial kernel scaffold; baseline (speedup 1.0000x reference)
import jax
import jax.numpy as jnp
from jax.experimental import pallas as pl


def kernel(edge_indices, features, W1, b1, W2, b2, W3, b3):
    raise NotImplementedError("write your pallas kernel here")


# speedup vs baseline: 5.9038x; 5.9038x over previous
import functools

import jax
import jax.numpy as jnp
from jax import lax
from jax.experimental import pallas as pl
from jax.experimental.pallas import tpu as pltpu
from jax.experimental.pallas import tpu_sc as plsc

N = 10000
E = 160000
NP = 10240
STRIPE = NP // 16
B = 128
NB = (E // 16) // B
REM = E // 16 - NB * B


def _zero_vmem_rows(ref, nrows, width):
    z = jnp.zeros((16,), jnp.float32)

    def body(i, _):
        for m in range(width // 16):
            ref[i, pl.ds(m * 16, 16)] = z
        return 0

    lax.fori_loop(0, nrows, body, 0)


def _fill_vmem_rows(ref, nrows, width, value):
    v = jnp.full((16,), value, jnp.float32)

    def body(i, _):
        for m in range(width // 16):
            ref[i, pl.ds(m * 16, 16)] = v
        return 0

    lax.fori_loop(0, nrows, body, 0)


def _deg_body(dst, out, zrow, onev, one16, d_v, d16, sem, accum):
    del sem
    c = lax.axis_index("c")
    s = lax.axis_index("s")

    @pl.when(c == 0)
    def _():
        _zero_vmem_rows(zrow, 128, 128)
        _fill_vmem_rows(onev, 128, 128, 1.0)
        _fill_vmem_rows(one16, REM, 128, 1.0)
        for j in range(STRIPE // 128):
            pltpu.sync_copy(zrow, accum.at[pl.ds(s * STRIPE + j * 128, 128), :])
        plsc.subcore_barrier()

        def batch(b, _):
            base = s * (E // 16) + b * B
            pltpu.sync_copy(dst.at[pl.ds(base, B)], d_v)
            pltpu.sync_copy(onev, accum.at[d_v], add=True)
            return 0

        lax.fori_loop(0, NB, batch, 0)
        rbase = s * (E // 16) + NB * B
        pltpu.sync_copy(dst.at[pl.ds(rbase, REM)], d16)
        pltpu.sync_copy(one16, accum.at[d16], add=True)
        plsc.subcore_barrier()
        pltpu.sync_copy(accum.at[pl.ds(s * STRIPE, STRIPE), :],
                        out.at[pl.ds(s * STRIPE, STRIPE), :])


_deg_kernel = pl.kernel(
    _deg_body,
    out_type=jax.ShapeDtypeStruct((NP, 128), jnp.float32),
    mesh=plsc.VectorSubcoreMesh(core_axis_name="c", subcore_axis_name="s"),
    scratch_types=[
        pltpu.VMEM((128, 128), jnp.float32),
        pltpu.VMEM((128, 128), jnp.float32),
        pltpu.VMEM((REM, 128), jnp.float32),
        pltpu.VMEM((B,), jnp.int32),
        pltpu.VMEM((REM,), jnp.int32),
        pltpu.SemaphoreType.DMA,
        pltpu.VMEM_SHARED((NP, 128), jnp.float32),
    ],
)


def _agg_body(nslab, hs, src, dst, out, zrow, s_v, g_v, d_v, s16, g16, d16,
              rows, rows16, sem, accum):
    c = lax.axis_index("c")
    s = lax.axis_index("s")
    spc = nslab // 2
    _zero_vmem_rows(zrow, 128, 128)

    for k in range(spc):
        slab = c * spc + k
        off = slab * N
        for j in range(STRIPE // 128):
            pltpu.sync_copy(zrow, accum.at[pl.ds(s * STRIPE + j * 128, 128), :])
        plsc.subcore_barrier()

        def batch(b, _):
            base = s * (E // 16) + b * B
            pltpu.sync_copy(src.at[pl.ds(base, B)], s_v)
            pltpu.sync_copy(dst.at[pl.ds(base, B)], d_v)
            for m in range(B // 16):
                g_v[pl.ds(m * 16, 16)] = s_v[pl.ds(m * 16, 16)] + off
            pltpu.async_copy(hs.at[g_v], rows, sem).wait()
            pltpu.sync_copy(rows, accum.at[d_v], add=True)
            return 0

        lax.fori_loop(0, NB, batch, 0)
        rbase = s * (E // 16) + NB * B
        pltpu.sync_copy(src.at[pl.ds(rbase, REM)], s16)
        pltpu.sync_copy(dst.at[pl.ds(rbase, REM)], d16)
        g16[pl.ds(0, 16)] = s16[pl.ds(0, 16)] + off
        pltpu.async_copy(hs.at[g16], rows16, sem).wait()
        pltpu.sync_copy(rows16, accum.at[d16], add=True)
        plsc.subcore_barrier()

        @pl.when(s < 15)
        def _():
            pltpu.sync_copy(
                accum.at[pl.ds(s * STRIPE, STRIPE), :],
                out.at[pl.ds(s * STRIPE, STRIPE), pl.ds(slab * 128, 128)])

        @pl.when(s == 15)
        def _():
            pltpu.sync_copy(
                accum.at[pl.ds(15 * STRIPE, N - 15 * STRIPE), :],
                out.at[pl.ds(15 * STRIPE, N - 15 * STRIPE),
                       pl.ds(slab * 128, 128)])


@functools.cache
def _make_agg(nslab):
    return pl.kernel(
        functools.partial(_agg_body, nslab),
        out_type=jax.ShapeDtypeStruct((N, nslab * 128), jnp.float32),
        mesh=plsc.VectorSubcoreMesh(core_axis_name="c", subcore_axis_name="s"),
        scratch_types=[
            pltpu.VMEM((128, 128), jnp.float32),
            pltpu.VMEM((B,), jnp.int32),
            pltpu.VMEM((B,), jnp.int32),
            pltpu.VMEM((B,), jnp.int32),
            pltpu.VMEM((REM,), jnp.int32),
            pltpu.VMEM((REM,), jnp.int32),
            pltpu.VMEM((REM,), jnp.int32),
            pltpu.VMEM((B, 128), jnp.float32),
            pltpu.VMEM((REM, 128), jnp.float32),
            pltpu.SemaphoreType.DMA,
            pltpu.VMEM_SHARED((NP, 128), jnp.float32),
        ],
    )


def _prep_body(deg_ref, out_ref):
    d = deg_ref[:, 0:1] + 1.0
    out_ref[...] = jnp.broadcast_to(lax.rsqrt(d), out_ref.shape)


def _prep(degp):
    return pl.pallas_call(
        _prep_body,
        grid=(NP // 512,),
        in_specs=[pl.BlockSpec((512, 128), lambda i: (i, 0))],
        out_specs=pl.BlockSpec((512, 128), lambda i: (i, 0)),
        out_shape=jax.ShapeDtypeStruct((NP, 128), jnp.float32),
    )(degp)


def _mm_body(x_ref, w_ref, dis_ref, hs_n_ref, hs_s_ref):
    h = jnp.dot(x_ref[...], w_ref[...],
                preferred_element_type=jnp.float32,
                precision=lax.Precision.HIGHEST)
    hs = h * dis_ref[...]
    hs_n_ref[...] = hs
    hs_s_ref[...] = hs[None]


def _mm(x, w, dis_bc):
    fin, fout = w.shape
    nslab = fout // 128
    return pl.pallas_call(
        _mm_body,
        grid=(N // 400, nslab),
        in_specs=[
            pl.BlockSpec((400, fin), lambda i, j: (i, 0)),
            pl.BlockSpec((fin, 128), lambda i, j: (0, j)),
            pl.BlockSpec((400, 128), lambda i, j: (i, 0)),
        ],
        out_specs=[
            pl.BlockSpec((400, 128), lambda i, j: (i, j)),
            pl.BlockSpec((1, 400, 128), lambda i, j: (j, i, 0)),
        ],
        out_shape=[
            jax.ShapeDtypeStruct((N, fout), jnp.float32),
            jax.ShapeDtypeStruct((nslab, N, 128), jnp.float32),
        ],
    )(x, w, dis_bc)


def _elem_body(relu, agg_ref, hs_ref, dis_ref, b_ref, out_ref):
    v = dis_ref[...] * (agg_ref[...] + hs_ref[...]) + b_ref[...]
    if relu:
        v = jnp.maximum(v, 0.0)
    out_ref[...] = v


def _elem(agg, hs_n, dis_bc, bias, relu):
    f = agg.shape[1]
    return pl.pallas_call(
        functools.partial(_elem_body, relu),
        grid=(N // 400, f // 128),
        in_specs=[
            pl.BlockSpec((400, 128), lambda i, j: (i, j)),
            pl.BlockSpec((400, 128), lambda i, j: (i, j)),
            pl.BlockSpec((400, 128), lambda i, j: (i, 0)),
            pl.BlockSpec((1, 128), lambda i, j: (0, j)),
        ],
        out_specs=pl.BlockSpec((400, 128), lambda i, j: (i, j)),
        out_shape=jax.ShapeDtypeStruct((N, f), jnp.float32),
    )(agg, hs_n, dis_bc, bias.reshape(1, f))


def _layer(x, w, b, src, dst, dis_bc, relu):
    hs_n, hs_s = _mm(x, w, dis_bc)
    nslab = w.shape[1] // 128
    agg = _make_agg(nslab)(hs_s.reshape(nslab * N, 128), src, dst)
    return _elem(agg, hs_n, dis_bc, b, relu)


def kernel(edge_indices, features, W1, b1, W2, b2, W3, b3):
    edge = jnp.asarray(edge_indices, jnp.int32)
    src, dst = edge[0], edge[1]
    degp = _deg_kernel(dst)
    dis_bc = _prep(degp)
    h = _layer(features, W1, b1, src, dst, dis_bc, True)
    h = _layer(h, W2, b2, src, dst, dis_bc, True)
    return _layer(h, W3, b3, src, dst, dis_bc, False)


# speedup vs baseline: 8.7715x; 1.4857x over previous
import functools

import jax
import jax.numpy as jnp
from jax import lax
from jax.experimental import pallas as pl
from jax.experimental.pallas import tpu as pltpu
from jax.experimental.pallas import tpu_sc as plsc

N = 10000
E = 160000
NP = 10240
STRIPE = NP // 16
B = 128
NB = (E // 16) // B
REM = E // 16 - NB * B


def _zero_vmem_rows(ref, nrows, width):
    z = jnp.zeros((16,), jnp.float32)

    def body(i, _):
        for m in range(width // 16):
            ref[i, pl.ds(m * 16, 16)] = z
        return 0

    lax.fori_loop(0, nrows, body, 0)


def _fill_vmem_rows(ref, nrows, width, value):
    v = jnp.full((16,), value, jnp.float32)

    def body(i, _):
        for m in range(width // 16):
            ref[i, pl.ds(m * 16, 16)] = v
        return 0

    lax.fori_loop(0, nrows, body, 0)


def _deg_body(dst, out, zrow, onev, one16, d_v, d16, sem, accum):
    del sem
    c = lax.axis_index("c")
    s = lax.axis_index("s")

    @pl.when(c == 0)
    def _():
        _zero_vmem_rows(zrow, 64, 128)
        _fill_vmem_rows(onev, 128, 128, 1.0)
        _fill_vmem_rows(one16, REM, 128, 1.0)
        for j in range(STRIPE // 64):
            pltpu.sync_copy(zrow, accum.at[pl.ds(s * STRIPE + j * 64, 64), :])
        plsc.subcore_barrier()

        def batch(b, _):
            base = s * (E // 16) + b * B
            pltpu.sync_copy(dst.at[pl.ds(base, B)], d_v)
            pltpu.sync_copy(onev, accum.at[d_v], add=True)
            return 0

        lax.fori_loop(0, NB, batch, 0)
        rbase = s * (E // 16) + NB * B
        pltpu.sync_copy(dst.at[pl.ds(rbase, REM)], d16)
        pltpu.sync_copy(one16, accum.at[d16], add=True)
        plsc.subcore_barrier()
        pltpu.sync_copy(accum.at[pl.ds(s * STRIPE, STRIPE), :],
                        out.at[pl.ds(s * STRIPE, STRIPE), :])


_deg_kernel = pl.kernel(
    _deg_body,
    out_type=jax.ShapeDtypeStruct((NP, 128), jnp.float32),
    mesh=plsc.VectorSubcoreMesh(core_axis_name="c", subcore_axis_name="s"),
    scratch_types=[
        pltpu.VMEM((64, 128), jnp.float32),
        pltpu.VMEM((128, 128), jnp.float32),
        pltpu.VMEM((REM, 128), jnp.float32),
        pltpu.VMEM((B,), jnp.int32),
        pltpu.VMEM((REM,), jnp.int32),
        pltpu.SemaphoreType.DMA,
        pltpu.VMEM_SHARED((NP, 128), jnp.float32),
    ],
)


def _agg_body(nslab, hs, src, dst, out, zrow, s_v0, g_v0, d_v0, s_v1, g_v1,
              d_v1, s16, g16, d16, rows0, rows1, rows16, sem_g0, sem_g1, sem,
              accum):
    c = lax.axis_index("c")
    s = lax.axis_index("s")
    spc = nslab // 2
    _zero_vmem_rows(zrow, 64, 128)
    sv = (s_v0, s_v1)
    gv = (g_v0, g_v1)
    dv = (d_v0, d_v1)
    rows = (rows0, rows1)
    semg = (sem_g0, sem_g1)
    ebase = s * (E // 16)

    for k in range(spc):
        slab = c * spc + k
        off = slab * N
        for j in range(STRIPE // 64):
            pltpu.sync_copy(zrow, accum.at[pl.ds(s * STRIPE + j * 64, 64), :])
        plsc.subcore_barrier()

        rbase = ebase + NB * B
        pltpu.sync_copy(src.at[pl.ds(rbase, REM)], s16)
        pltpu.sync_copy(dst.at[pl.ds(rbase, REM)], d16)
        g16[pl.ds(0, 16)] = s16[pl.ds(0, 16)] + off
        pltpu.async_copy(hs.at[g16], rows16, sem).wait()
        pltpu.sync_copy(rows16, accum.at[d16], add=True)

        pltpu.sync_copy(src.at[pl.ds(ebase, B)], s_v0)
        pltpu.sync_copy(dst.at[pl.ds(ebase, B)], d_v0)
        for m in range(B // 16):
            g_v0[pl.ds(m * 16, 16)] = s_v0[pl.ds(m * 16, 16)] + off
        pltpu.async_copy(hs.at[g_v0], rows0, sem_g0)

        def pair(t, _):
            for p in (0, 1):
                b = 2 * t + p
                q = 1 - p
                nb = b + 1

                @pl.when(nb < NB)
                def _():
                    base = ebase + nb * B
                    pltpu.sync_copy(src.at[pl.ds(base, B)], sv[q])
                    pltpu.sync_copy(dst.at[pl.ds(base, B)], dv[q])
                    for m in range(B // 16):
                        gv[q][pl.ds(m * 16, 16)] = (
                            sv[q][pl.ds(m * 16, 16)] + off)
                    pltpu.async_copy(hs.at[gv[q]], rows[q], semg[q])

                pltpu.make_async_copy(hs.at[gv[p]], rows[p], semg[p]).wait()
                pltpu.sync_copy(rows[p], accum.at[dv[p]], add=True)
            return 0

        lax.fori_loop(0, NB // 2, pair, 0)
        plsc.subcore_barrier()

        @pl.when(s < 15)
        def _():
            pltpu.sync_copy(
                accum.at[pl.ds(s * STRIPE, STRIPE), :],
                out.at[pl.ds(s * STRIPE, STRIPE), pl.ds(slab * 128, 128)])

        @pl.when(s == 15)
        def _():
            pltpu.sync_copy(
                accum.at[pl.ds(15 * STRIPE, N - 15 * STRIPE), :],
                out.at[pl.ds(15 * STRIPE, N - 15 * STRIPE),
                       pl.ds(slab * 128, 128)])


@functools.cache
def _make_agg(nslab):
    return pl.kernel(
        functools.partial(_agg_body, nslab),
        out_type=jax.ShapeDtypeStruct((N, nslab * 128), jnp.float32),
        mesh=plsc.VectorSubcoreMesh(core_axis_name="c", subcore_axis_name="s"),
        scratch_types=[
            pltpu.VMEM((64, 128), jnp.float32),
            pltpu.VMEM((B,), jnp.int32),
            pltpu.VMEM((B,), jnp.int32),
            pltpu.VMEM((B,), jnp.int32),
            pltpu.VMEM((B,), jnp.int32),
            pltpu.VMEM((B,), jnp.int32),
            pltpu.VMEM((B,), jnp.int32),
            pltpu.VMEM((REM,), jnp.int32),
            pltpu.VMEM((REM,), jnp.int32),
            pltpu.VMEM((REM,), jnp.int32),
            pltpu.VMEM((B, 128), jnp.float32),
            pltpu.VMEM((B, 128), jnp.float32),
            pltpu.VMEM((REM, 128), jnp.float32),
            pltpu.SemaphoreType.DMA,
            pltpu.SemaphoreType.DMA,
            pltpu.SemaphoreType.DMA,
            pltpu.VMEM_SHARED((NP, 128), jnp.float32),
        ],
    )


def _prep_body(deg_ref, out_ref):
    d = deg_ref[:, 0:1] + 1.0
    out_ref[...] = jnp.broadcast_to(lax.rsqrt(d), out_ref.shape)


def _prep(degp):
    return pl.pallas_call(
        _prep_body,
        grid=(NP // 512,),
        in_specs=[pl.BlockSpec((512, 128), lambda i: (i, 0))],
        out_specs=pl.BlockSpec((512, 128), lambda i: (i, 0)),
        out_shape=jax.ShapeDtypeStruct((NP, 128), jnp.float32),
    )(degp)


def _mm_body(x_ref, w_ref, dis_ref, hs_n_ref, hs_s_ref):
    h = jnp.dot(x_ref[...], w_ref[...],
                preferred_element_type=jnp.float32,
                precision=lax.Precision.HIGHEST)
    hs = h * dis_ref[...]
    hs_n_ref[...] = hs
    hs_s_ref[...] = hs[None]


def _mm(x, w, dis_bc):
    fin, fout = w.shape
    nslab = fout // 128
    return pl.pallas_call(
        _mm_body,
        grid=(N // 400, nslab),
        in_specs=[
            pl.BlockSpec((400, fin), lambda i, j: (i, 0)),
            pl.BlockSpec((fin, 128), lambda i, j: (0, j)),
            pl.BlockSpec((400, 128), lambda i, j: (i, 0)),
        ],
        out_specs=[
            pl.BlockSpec((400, 128), lambda i, j: (i, j)),
            pl.BlockSpec((1, 400, 128), lambda i, j: (j, i, 0)),
        ],
        out_shape=[
            jax.ShapeDtypeStruct((N, fout), jnp.float32),
            jax.ShapeDtypeStruct((nslab, N, 128), jnp.float32),
        ],
    )(x, w, dis_bc)


def _mmf_body(agg_ref, hsp_ref, bp_ref, w_ref, dis_ref, hs_n_ref, hs_s_ref):
    d = dis_ref[:, 0:1]
    x = jnp.maximum(d * (agg_ref[...] + hsp_ref[...]) + bp_ref[...], 0.0)
    h = jnp.dot(x, w_ref[...],
                preferred_element_type=jnp.float32,
                precision=lax.Precision.HIGHEST)
    hs_n_ref[...] = h * dis_ref[...]
    hs_s_ref[...] = (h * dis_ref[...])[None]


def _mm_fused(agg, hs_p, b_p, w, dis_bc):
    fin, fout = w.shape
    nslab = fout // 128
    return pl.pallas_call(
        _mmf_body,
        grid=(N // 400, nslab),
        in_specs=[
            pl.BlockSpec((400, fin), lambda i, j: (i, 0)),
            pl.BlockSpec((400, fin), lambda i, j: (i, 0)),
            pl.BlockSpec((1, fin), lambda i, j: (0, 0)),
            pl.BlockSpec((fin, 128), lambda i, j: (0, j)),
            pl.BlockSpec((400, 128), lambda i, j: (i, 0)),
        ],
        out_specs=[
            pl.BlockSpec((400, 128), lambda i, j: (i, j)),
            pl.BlockSpec((1, 400, 128), lambda i, j: (j, i, 0)),
        ],
        out_shape=[
            jax.ShapeDtypeStruct((N, fout), jnp.float32),
            jax.ShapeDtypeStruct((nslab, N, 128), jnp.float32),
        ],
    )(agg, hs_p, b_p.reshape(1, fin), w, dis_bc)


def _elem_body(relu, agg_ref, hs_ref, dis_ref, b_ref, out_ref):
    v = dis_ref[...] * (agg_ref[...] + hs_ref[...]) + b_ref[...]
    if relu:
        v = jnp.maximum(v, 0.0)
    out_ref[...] = v


def _elem(agg, hs_n, dis_bc, bias, relu):
    f = agg.shape[1]
    return pl.pallas_call(
        functools.partial(_elem_body, relu),
        grid=(N // 400, f // 128),
        in_specs=[
            pl.BlockSpec((400, 128), lambda i, j: (i, j)),
            pl.BlockSpec((400, 128), lambda i, j: (i, j)),
            pl.BlockSpec((400, 128), lambda i, j: (i, 0)),
            pl.BlockSpec((1, 128), lambda i, j: (0, j)),
        ],
        out_specs=pl.BlockSpec((400, 128), lambda i, j: (i, j)),
        out_shape=jax.ShapeDtypeStruct((N, f), jnp.float32),
    )(agg, hs_n, dis_bc, bias.reshape(1, f))


def _agg_of(hs_s, src, dst):
    nslab = hs_s.shape[0]
    return _make_agg(nslab)(hs_s.reshape(nslab * N, 128), src, dst)


def kernel(edge_indices, features, W1, b1, W2, b2, W3, b3):
    edge = jnp.asarray(edge_indices, jnp.int32)
    src, dst = edge[0], edge[1]
    degp = _deg_kernel(dst)
    dis_bc = _prep(degp)
    hs_n1, hs_s1 = _mm(features, W1, dis_bc)
    agg1 = _agg_of(hs_s1, src, dst)
    hs_n2, hs_s2 = _mm_fused(agg1, hs_n1, b1, W2, dis_bc)
    agg2 = _agg_of(hs_s2, src, dst)
    hs_n3, hs_s3 = _mm_fused(agg2, hs_n2, b2, W3, dis_bc)
    agg3 = _agg_of(hs_s3, src, dst)
    return _elem(agg3, hs_n3, dis_bc, b3, False)


# speedup vs baseline: 9.6056x; 1.0951x over previous
import functools

import jax
import jax.numpy as jnp
from jax import lax
from jax.experimental import pallas as pl
from jax.experimental.pallas import tpu as pltpu
from jax.experimental.pallas import tpu_sc as plsc

N = 10000
E = 160000
NP = 10240
STRIPE = NP // 16
B = 128
NB = (E // 16) // B
REM = E // 16 - NB * B


def _zero_vmem_rows(ref, nrows, width):
    z = jnp.zeros((16,), jnp.float32)

    def body(i, _):
        for m in range(width // 16):
            ref[i, pl.ds(m * 16, 16)] = z
        return 0

    lax.fori_loop(0, nrows, body, 0)


def _fill_vmem_rows(ref, nrows, width, value):
    v = jnp.full((16,), value, jnp.float32)

    def body(i, _):
        for m in range(width // 16):
            ref[i, pl.ds(m * 16, 16)] = v
        return 0

    lax.fori_loop(0, nrows, body, 0)


def _deg_body(dst, out, zrow, onev, one16, d_v, d16, sem, accum):
    del sem
    c = lax.axis_index("c")
    s = lax.axis_index("s")
    _zero_vmem_rows(zrow, 64, 128)
    _fill_vmem_rows(onev, 128, 128, 1.0)
    _fill_vmem_rows(one16, REM, 128, 1.0)
    for j in range(STRIPE // 64):
        pltpu.sync_copy(zrow, accum.at[pl.ds(s * STRIPE + j * 64, 64), :])
    plsc.subcore_barrier()

    def batch(t, _):
        base = s * (E // 16) + (2 * t + c) * B
        pltpu.sync_copy(dst.at[pl.ds(base, B)], d_v)
        pltpu.sync_copy(onev, accum.at[d_v], add=True)
        return 0

    lax.fori_loop(0, NB // 2, batch, 0)

    @pl.when(c == 0)
    def _():
        rbase = s * (E // 16) + NB * B
        pltpu.sync_copy(dst.at[pl.ds(rbase, REM)], d16)
        pltpu.sync_copy(one16, accum.at[d16], add=True)

    plsc.subcore_barrier()
    pltpu.sync_copy(accum.at[pl.ds(s * STRIPE, STRIPE), :],
                    out.at[c, pl.ds(s * STRIPE, STRIPE), :])


_deg_kernel = pl.kernel(
    _deg_body,
    out_type=jax.ShapeDtypeStruct((2, NP, 128), jnp.float32),
    mesh=plsc.VectorSubcoreMesh(core_axis_name="c", subcore_axis_name="s"),
    scratch_types=[
        pltpu.VMEM((64, 128), jnp.float32),
        pltpu.VMEM((128, 128), jnp.float32),
        pltpu.VMEM((REM, 128), jnp.float32),
        pltpu.VMEM((B,), jnp.int32),
        pltpu.VMEM((REM,), jnp.int32),
        pltpu.SemaphoreType.DMA,
        pltpu.VMEM_SHARED((NP, 128), jnp.float32),
    ],
)


def _agg_body(nslab, hs, src, dst, out, zrow, s_v0, g_v0, d_v0, s_v1, g_v1,
              d_v1, s16, g16, d16, rows0, rows1, rows16, sem_g0, sem_g1,
              sem_s0, sem_s1, sem, accum):
    c = lax.axis_index("c")
    s = lax.axis_index("s")
    spc = nslab // 2
    _zero_vmem_rows(zrow, 64, 128)
    sv = (s_v0, s_v1)
    gv = (g_v0, g_v1)
    dv = (d_v0, d_v1)
    rows = (rows0, rows1)
    semg = (sem_g0, sem_g1)
    sems = (sem_s0, sem_s1)
    ebase = s * (E // 16)

    for k in range(spc):
        slab = c * spc + k
        off = slab * N
        for j in range(STRIPE // 64):
            pltpu.sync_copy(zrow, accum.at[pl.ds(s * STRIPE + j * 64, 64), :])
        plsc.subcore_barrier()

        rbase = ebase + NB * B
        pltpu.sync_copy(src.at[pl.ds(rbase, REM)], s16)
        pltpu.sync_copy(dst.at[pl.ds(rbase, REM)], d16)
        g16[pl.ds(0, 16)] = s16[pl.ds(0, 16)] + off
        pltpu.async_copy(hs.at[g16], rows16, sem).wait()
        pltpu.sync_copy(rows16, accum.at[d16], add=True)

        pltpu.sync_copy(src.at[pl.ds(ebase, B)], s_v0)
        pltpu.sync_copy(dst.at[pl.ds(ebase, B)], d_v0)
        for m in range(B // 16):
            g_v0[pl.ds(m * 16, 16)] = s_v0[pl.ds(m * 16, 16)] + off
        pltpu.async_copy(hs.at[g_v0], rows0, sem_g0)

        def pair(t, _):
            for p in (0, 1):
                b = 2 * t + p
                q = 1 - p
                nb = b + 1

                @pl.when(b >= 1)
                def _():
                    pltpu.make_async_copy(
                        rows[q], accum.at[dv[q]], sems[q]).wait()

                @pl.when(nb < NB)
                def _():
                    base = ebase + nb * B
                    pltpu.sync_copy(src.at[pl.ds(base, B)], sv[q])
                    pltpu.sync_copy(dst.at[pl.ds(base, B)], dv[q])
                    for m in range(B // 16):
                        gv[q][pl.ds(m * 16, 16)] = (
                            sv[q][pl.ds(m * 16, 16)] + off)
                    pltpu.async_copy(hs.at[gv[q]], rows[q], semg[q])

                pltpu.make_async_copy(hs.at[gv[p]], rows[p], semg[p]).wait()
                pltpu.async_copy(rows[p], accum.at[dv[p]], sems[p], add=True)
            return 0

        lax.fori_loop(0, NB // 2, pair, 0)
        pltpu.make_async_copy(rows[1], accum.at[dv[1]], sems[1]).wait()
        plsc.subcore_barrier()

        @pl.when(s < 15)
        def _():
            pltpu.sync_copy(
                accum.at[pl.ds(s * STRIPE, STRIPE), :],
                out.at[pl.ds(s * STRIPE, STRIPE), pl.ds(slab * 128, 128)])

        @pl.when(s == 15)
        def _():
            pltpu.sync_copy(
                accum.at[pl.ds(15 * STRIPE, N - 15 * STRIPE), :],
                out.at[pl.ds(15 * STRIPE, N - 15 * STRIPE),
                       pl.ds(slab * 128, 128)])


@functools.cache
def _make_agg(nslab):
    return pl.kernel(
        functools.partial(_agg_body, nslab),
        out_type=jax.ShapeDtypeStruct((N, nslab * 128), jnp.float32),
        mesh=plsc.VectorSubcoreMesh(core_axis_name="c", subcore_axis_name="s"),
        scratch_types=[
            pltpu.VMEM((64, 128), jnp.float32),
            pltpu.VMEM((B,), jnp.int32),
            pltpu.VMEM((B,), jnp.int32),
            pltpu.VMEM((B,), jnp.int32),
            pltpu.VMEM((B,), jnp.int32),
            pltpu.VMEM((B,), jnp.int32),
            pltpu.VMEM((B,), jnp.int32),
            pltpu.VMEM((REM,), jnp.int32),
            pltpu.VMEM((REM,), jnp.int32),
            pltpu.VMEM((REM,), jnp.int32),
            pltpu.VMEM((B, 128), jnp.float32),
            pltpu.VMEM((B, 128), jnp.float32),
            pltpu.VMEM((REM, 128), jnp.float32),
            pltpu.SemaphoreType.DMA,
            pltpu.SemaphoreType.DMA,
            pltpu.SemaphoreType.DMA,
            pltpu.SemaphoreType.DMA,
            pltpu.SemaphoreType.DMA,
            pltpu.VMEM_SHARED((NP, 128), jnp.float32),
        ],
    )


def _prep_body(deg_ref, out_ref):
    d = deg_ref[0, :, 0:1] + deg_ref[1, :, 0:1] + 1.0
    out_ref[...] = jnp.broadcast_to(lax.rsqrt(d), out_ref.shape)


def _prep(degp):
    return pl.pallas_call(
        _prep_body,
        grid=(NP // 512,),
        in_specs=[pl.BlockSpec((2, 512, 128), lambda i: (0, i, 0))],
        out_specs=pl.BlockSpec((512, 128), lambda i: (i, 0)),
        out_shape=jax.ShapeDtypeStruct((NP, 128), jnp.float32),
    )(degp)


def _mm_body(x_ref, w_ref, dis_ref, hs_n_ref, hs_s_ref):
    h = jnp.dot(x_ref[...], w_ref[...],
                preferred_element_type=jnp.float32,
                precision=lax.Precision.DEFAULT)
    hs = h * dis_ref[...]
    hs_n_ref[...] = hs
    hs_s_ref[...] = hs[None]


def _mm(x, w, dis_bc):
    fin, fout = w.shape
    nslab = fout // 128
    return pl.pallas_call(
        _mm_body,
        grid=(N // 400, nslab),
        in_specs=[
            pl.BlockSpec((400, fin), lambda i, j: (i, 0)),
            pl.BlockSpec((fin, 128), lambda i, j: (0, j)),
            pl.BlockSpec((400, 128), lambda i, j: (i, 0)),
        ],
        out_specs=[
            pl.BlockSpec((400, 128), lambda i, j: (i, j)),
            pl.BlockSpec((1, 400, 128), lambda i, j: (j, i, 0)),
        ],
        out_shape=[
            jax.ShapeDtypeStruct((N, fout), jnp.float32),
            jax.ShapeDtypeStruct((nslab, N, 128), jnp.float32),
        ],
    )(x, w, dis_bc)


def _mmf_body(agg_ref, hsp_ref, bp_ref, w_ref, dis_ref, hs_n_ref, hs_s_ref):
    d = dis_ref[:, 0:1]
    x = jnp.maximum(d * (agg_ref[...] + hsp_ref[...]) + bp_ref[...], 0.0)
    h = jnp.dot(x, w_ref[...],
                preferred_element_type=jnp.float32,
                precision=lax.Precision.DEFAULT)
    hs_n_ref[...] = h * dis_ref[...]
    hs_s_ref[...] = (h * dis_ref[...])[None]


def _mm_fused(agg, hs_p, b_p, w, dis_bc):
    fin, fout = w.shape
    nslab = fout // 128
    return pl.pallas_call(
        _mmf_body,
        grid=(N // 400, nslab),
        in_specs=[
            pl.BlockSpec((400, fin), lambda i, j: (i, 0)),
            pl.BlockSpec((400, fin), lambda i, j: (i, 0)),
            pl.BlockSpec((1, fin), lambda i, j: (0, 0)),
            pl.BlockSpec((fin, 128), lambda i, j: (0, j)),
            pl.BlockSpec((400, 128), lambda i, j: (i, 0)),
        ],
        out_specs=[
            pl.BlockSpec((400, 128), lambda i, j: (i, j)),
            pl.BlockSpec((1, 400, 128), lambda i, j: (j, i, 0)),
        ],
        out_shape=[
            jax.ShapeDtypeStruct((N, fout), jnp.float32),
            jax.ShapeDtypeStruct((nslab, N, 128), jnp.float32),
        ],
    )(agg, hs_p, b_p.reshape(1, fin), w, dis_bc)


def _elem_body(relu, agg_ref, hs_ref, dis_ref, b_ref, out_ref):
    v = dis_ref[...] * (agg_ref[...] + hs_ref[...]) + b_ref[...]
    if relu:
        v = jnp.maximum(v, 0.0)
    out_ref[...] = v


def _elem(agg, hs_n, dis_bc, bias, relu):
    f = agg.shape[1]
    return pl.pallas_call(
        functools.partial(_elem_body, relu),
        grid=(N // 400, f // 128),
        in_specs=[
            pl.BlockSpec((400, 128), lambda i, j: (i, j)),
            pl.BlockSpec((400, 128), lambda i, j: (i, j)),
            pl.BlockSpec((400, 128), lambda i, j: (i, 0)),
            pl.BlockSpec((1, 128), lambda i, j: (0, j)),
        ],
        out_specs=pl.BlockSpec((400, 128), lambda i, j: (i, j)),
        out_shape=jax.ShapeDtypeStruct((N, f), jnp.float32),
    )(agg, hs_n, dis_bc, bias.reshape(1, f))


def _agg_of(hs_s, src, dst):
    nslab = hs_s.shape[0]
    return _make_agg(nslab)(hs_s.reshape(nslab * N, 128), src, dst)


def kernel(edge_indices, features, W1, b1, W2, b2, W3, b3):
    edge = jnp.asarray(edge_indices, jnp.int32)
    src, dst = edge[0], edge[1]
    degp = _deg_kernel(dst)
    dis_bc = _prep(degp)
    hs_n1, hs_s1 = _mm(features, W1, dis_bc)
    agg1 = _agg_of(hs_s1, src, dst)
    hs_n2, hs_s2 = _mm_fused(agg1, hs_n1, b1, W2, dis_bc)
    agg2 = _agg_of(hs_s2, src, dst)
    hs_n3, hs_s3 = _mm_fused(agg2, hs_n2, b2, W3, dis_bc)
    agg3 = _agg_of(hs_s3, src, dst)
    return _elem(agg3, hs_n3, dis_bc, b3, False)


# speedup vs baseline: 11.5569x; 1.2031x over previous
import functools

import jax
import jax.numpy as jnp
from jax import lax
from jax.experimental import pallas as pl
from jax.experimental.pallas import tpu as pltpu
from jax.experimental.pallas import tpu_sc as plsc

N = 10000
E = 160000
NP = 10240
STRIPE = NP // 16
B = 128
NB = (E // 16) // B
REM = E // 16 - NB * B


def _zero_vmem_rows(ref, nrows, width):
    z = jnp.zeros((16,), jnp.float32)

    def body(i, _):
        for m in range(width // 16):
            ref[i, pl.ds(m * 16, 16)] = z
        return 0

    lax.fori_loop(0, nrows, body, 0)


def _fill_vmem_rows(ref, nrows, width, value):
    v = jnp.full((16,), value, jnp.float32)

    def body(i, _):
        for m in range(width // 16):
            ref[i, pl.ds(m * 16, 16)] = v
        return 0

    lax.fori_loop(0, nrows, body, 0)


def _deg_body(dst, out, zrow, onev, one16, d_v, d16, sem, accum):
    del sem
    c = lax.axis_index("c")
    s = lax.axis_index("s")
    _zero_vmem_rows(zrow, 64, 128)
    _fill_vmem_rows(onev, 128, 128, 1.0)
    _fill_vmem_rows(one16, REM, 128, 1.0)
    for j in range(STRIPE // 64):
        pltpu.sync_copy(zrow, accum.at[pl.ds(s * STRIPE + j * 64, 64), :])
    plsc.subcore_barrier()

    def batch(t, _):
        base = s * (E // 16) + (2 * t + c) * B
        pltpu.sync_copy(dst.at[pl.ds(base, B)], d_v)
        pltpu.sync_copy(onev, accum.at[d_v], add=True)
        return 0

    lax.fori_loop(0, NB // 2, batch, 0)

    @pl.when(c == 0)
    def _():
        rbase = s * (E // 16) + NB * B
        pltpu.sync_copy(dst.at[pl.ds(rbase, REM)], d16)
        pltpu.sync_copy(one16, accum.at[d16], add=True)

    plsc.subcore_barrier()
    pltpu.sync_copy(accum.at[pl.ds(s * STRIPE, STRIPE), :],
                    out.at[c, pl.ds(s * STRIPE, STRIPE), :])


_deg_kernel = pl.kernel(
    _deg_body,
    out_type=jax.ShapeDtypeStruct((2, NP, 128), jnp.float32),
    mesh=plsc.VectorSubcoreMesh(core_axis_name="c", subcore_axis_name="s"),
    scratch_types=[
        pltpu.VMEM((64, 128), jnp.float32),
        pltpu.VMEM((128, 128), jnp.float32),
        pltpu.VMEM((REM, 128), jnp.float32),
        pltpu.VMEM((B,), jnp.int32),
        pltpu.VMEM((REM,), jnp.int32),
        pltpu.SemaphoreType.DMA,
        pltpu.VMEM_SHARED((NP, 128), jnp.float32),
    ],
)


def _agg_body(nslab, hs, src, dst, out, zrow, s_v0, g_v0, d_v0, s_v1, g_v1,
              d_v1, s16, g16, d16, rows0, rows1, rows16, sem_g0, sem_g1,
              sem_s0, sem_s1, sem, accum):
    c = lax.axis_index("c")
    s = lax.axis_index("s")
    spc = nslab // 2
    _zero_vmem_rows(zrow, 64, 128)
    sv = (s_v0, s_v1)
    gv = (g_v0, g_v1)
    dv = (d_v0, d_v1)
    rows = (rows0, rows1)
    semg = (sem_g0, sem_g1)
    sems = (sem_s0, sem_s1)
    ebase = s * (E // 16)

    for k in range(spc):
        slab = c * spc + k
        off = slab * N
        for j in range(STRIPE // 64):
            pltpu.sync_copy(zrow, accum.at[pl.ds(s * STRIPE + j * 64, 64), :])
        plsc.subcore_barrier()

        rbase = ebase + NB * B
        pltpu.sync_copy(src.at[pl.ds(rbase, REM)], s16)
        pltpu.sync_copy(dst.at[pl.ds(rbase, REM)], d16)
        g16[pl.ds(0, 16)] = s16[pl.ds(0, 16)] + off
        pltpu.async_copy(hs.at[g16], rows16, sem).wait()
        pltpu.sync_copy(rows16, accum.at[d16], add=True)

        pltpu.sync_copy(src.at[pl.ds(ebase, B)], s_v0)
        pltpu.sync_copy(dst.at[pl.ds(ebase, B)], d_v0)
        for m in range(B // 16):
            g_v0[pl.ds(m * 16, 16)] = s_v0[pl.ds(m * 16, 16)] + off
        pltpu.async_copy(hs.at[g_v0], rows0, sem_g0)

        def pair(t, _):
            for p in (0, 1):
                b = 2 * t + p
                q = 1 - p
                nb = b + 1

                @pl.when(b >= 1)
                def _():
                    pltpu.make_async_copy(
                        rows[q], accum.at[dv[q]], sems[q]).wait()

                @pl.when(nb < NB)
                def _():
                    base = ebase + nb * B
                    pltpu.sync_copy(src.at[pl.ds(base, B)], sv[q])
                    pltpu.sync_copy(dst.at[pl.ds(base, B)], dv[q])
                    for m in range(B // 16):
                        gv[q][pl.ds(m * 16, 16)] = (
                            sv[q][pl.ds(m * 16, 16)] + off)
                    pltpu.async_copy(hs.at[gv[q]], rows[q], semg[q])

                pltpu.make_async_copy(hs.at[gv[p]], rows[p], semg[p]).wait()
                pltpu.async_copy(rows[p], accum.at[dv[p]], sems[p], add=True)
            return 0

        lax.fori_loop(0, NB // 2, pair, 0)
        pltpu.make_async_copy(rows[1], accum.at[dv[1]], sems[1]).wait()
        plsc.subcore_barrier()

        @pl.when(s < 15)
        def _():
            pltpu.sync_copy(
                accum.at[pl.ds(s * STRIPE, STRIPE), :],
                out.at[pl.ds(s * STRIPE, STRIPE), pl.ds(slab * 128, 128)])

        @pl.when(s == 15)
        def _():
            pltpu.sync_copy(
                accum.at[pl.ds(15 * STRIPE, N - 15 * STRIPE), :],
                out.at[pl.ds(15 * STRIPE, N - 15 * STRIPE),
                       pl.ds(slab * 128, 128)])


@functools.cache
def _make_agg(nslab):
    return pl.kernel(
        functools.partial(_agg_body, nslab),
        out_type=jax.ShapeDtypeStruct((N, nslab * 128), jnp.float32),
        mesh=plsc.VectorSubcoreMesh(core_axis_name="c", subcore_axis_name="s"),
        scratch_types=[
            pltpu.VMEM((64, 128), jnp.float32),
            pltpu.VMEM((B,), jnp.int32),
            pltpu.VMEM((B,), jnp.int32),
            pltpu.VMEM((B,), jnp.int32),
            pltpu.VMEM((B,), jnp.int32),
            pltpu.VMEM((B,), jnp.int32),
            pltpu.VMEM((B,), jnp.int32),
            pltpu.VMEM((REM,), jnp.int32),
            pltpu.VMEM((REM,), jnp.int32),
            pltpu.VMEM((REM,), jnp.int32),
            pltpu.VMEM((B, 128), jnp.float32),
            pltpu.VMEM((B, 128), jnp.float32),
            pltpu.VMEM((REM, 128), jnp.float32),
            pltpu.SemaphoreType.DMA,
            pltpu.SemaphoreType.DMA,
            pltpu.SemaphoreType.DMA,
            pltpu.SemaphoreType.DMA,
            pltpu.SemaphoreType.DMA,
            pltpu.VMEM_SHARED((NP, 128), jnp.float32),
        ],
    )


def _dis(degp_ref):
    return lax.rsqrt(degp_ref[0, :, 0:1] + degp_ref[1, :, 0:1] + 1.0)


def _scale_body(feat_ref, degp_ref, xs_n_ref, xs_s_ref):
    xs = feat_ref[...] * _dis(degp_ref)
    xs_n_ref[...] = xs
    xs_s_ref[...] = xs[None]


def _scale(features, degp):
    fin = features.shape[1]
    nslab = fin // 128
    return pl.pallas_call(
        _scale_body,
        grid=(N // 400, nslab),
        in_specs=[
            pl.BlockSpec((400, 128), lambda i, j: (i, j)),
            pl.BlockSpec((2, 400, 128), lambda i, j: (0, i, 0)),
        ],
        out_specs=[
            pl.BlockSpec((400, 128), lambda i, j: (i, j)),
            pl.BlockSpec((1, 400, 128), lambda i, j: (j, i, 0)),
        ],
        out_shape=[
            jax.ShapeDtypeStruct((N, fin), jnp.float32),
            jax.ShapeDtypeStruct((nslab, N, 128), jnp.float32),
        ],
    )(features, degp)


def _mm2_body(z_ref, xs_ref, b1_ref, w1_ref, w2_ref, degp_ref,
              hs_n_ref, hs_s_ref):
    d = _dis(degp_ref)
    y = d * (z_ref[...] + xs_ref[...])
    x2 = jnp.maximum(
        jnp.dot(y, w1_ref[...], preferred_element_type=jnp.float32)
        + b1_ref[...], 0.0)
    h = jnp.dot(x2, w2_ref[...], preferred_element_type=jnp.float32)
    hs_n_ref[...] = h * d
    hs_s_ref[...] = (h * d)[None]


def _mm2(z1, xs1, b1, w1, w2, degp):
    fin = w1.shape[0]
    fmid = w1.shape[1]
    fout = w2.shape[1]
    nslab = fout // 128
    return pl.pallas_call(
        _mm2_body,
        grid=(N // 400, nslab),
        in_specs=[
            pl.BlockSpec((400, fin), lambda i, j: (i, 0)),
            pl.BlockSpec((400, fin), lambda i, j: (i, 0)),
            pl.BlockSpec((1, fmid), lambda i, j: (0, 0)),
            pl.BlockSpec((fin, fmid), lambda i, j: (0, 0)),
            pl.BlockSpec((fmid, 128), lambda i, j: (0, j)),
            pl.BlockSpec((2, 400, 128), lambda i, j: (0, i, 0)),
        ],
        out_specs=[
            pl.BlockSpec((400, 128), lambda i, j: (i, j)),
            pl.BlockSpec((1, 400, 128), lambda i, j: (j, i, 0)),
        ],
        out_shape=[
            jax.ShapeDtypeStruct((N, fout), jnp.float32),
            jax.ShapeDtypeStruct((nslab, N, 128), jnp.float32),
        ],
    )(z1, xs1, b1.reshape(1, fmid), w1, w2, degp)


def _mmf_body(agg_ref, hsp_ref, bp_ref, w_ref, degp_ref, hs_n_ref, hs_s_ref):
    d = _dis(degp_ref)
    x = jnp.maximum(d * (agg_ref[...] + hsp_ref[...]) + bp_ref[...], 0.0)
    h = jnp.dot(x, w_ref[...], preferred_element_type=jnp.float32)
    hs_n_ref[...] = h * d
    hs_s_ref[...] = (h * d)[None]


def _mm_fused(agg, hs_p, b_p, w, degp):
    fin, fout = w.shape
    nslab = fout // 128
    return pl.pallas_call(
        _mmf_body,
        grid=(N // 400, nslab),
        in_specs=[
            pl.BlockSpec((400, fin), lambda i, j: (i, 0)),
            pl.BlockSpec((400, fin), lambda i, j: (i, 0)),
            pl.BlockSpec((1, fin), lambda i, j: (0, 0)),
            pl.BlockSpec((fin, 128), lambda i, j: (0, j)),
            pl.BlockSpec((2, 400, 128), lambda i, j: (0, i, 0)),
        ],
        out_specs=[
            pl.BlockSpec((400, 128), lambda i, j: (i, j)),
            pl.BlockSpec((1, 400, 128), lambda i, j: (j, i, 0)),
        ],
        out_shape=[
            jax.ShapeDtypeStruct((N, fout), jnp.float32),
            jax.ShapeDtypeStruct((nslab, N, 128), jnp.float32),
        ],
    )(agg, hs_p, b_p.reshape(1, fin), w, degp)


def _elem_body(agg_ref, hs_ref, degp_ref, b_ref, out_ref):
    out_ref[...] = (_dis(degp_ref) * (agg_ref[...] + hs_ref[...])
                    + b_ref[...])


def _elem(agg, hs_n, degp, bias):
    f = agg.shape[1]
    return pl.pallas_call(
        _elem_body,
        grid=(N // 400, f // 128),
        in_specs=[
            pl.BlockSpec((400, 128), lambda i, j: (i, j)),
            pl.BlockSpec((400, 128), lambda i, j: (i, j)),
            pl.BlockSpec((2, 400, 128), lambda i, j: (0, i, 0)),
            pl.BlockSpec((1, 128), lambda i, j: (0, j)),
        ],
        out_specs=pl.BlockSpec((400, 128), lambda i, j: (i, j)),
        out_shape=jax.ShapeDtypeStruct((N, f), jnp.float32),
    )(agg, hs_n, degp, bias.reshape(1, f))


def _agg_of(hs_s, src, dst):
    nslab = hs_s.shape[0]
    return _make_agg(nslab)(hs_s.reshape(nslab * N, 128), src, dst)


def kernel(edge_indices, features, W1, b1, W2, b2, W3, b3):
    edge = jnp.asarray(edge_indices, jnp.int32)
    src, dst = edge[0], edge[1]
    degp = _deg_kernel(dst)
    xs1_n, xs1_s = _scale(features, degp)
    z1 = _agg_of(xs1_s, src, dst)
    hs_n2, hs_s2 = _mm2(z1, xs1_n, b1, W1, W2, degp)
    agg2 = _agg_of(hs_s2, src, dst)
    hs_n3, hs_s3 = _mm_fused(agg2, hs_n2, b2, W3, degp)
    agg3 = _agg_of(hs_s3, src, dst)
    return _elem(agg3, hs_n3, degp, b3)


# speedup vs baseline: 12.6939x; 1.0984x over previous
import functools

import jax
import jax.numpy as jnp
from jax import lax
from jax.experimental import pallas as pl
from jax.experimental.pallas import tpu as pltpu
from jax.experimental.pallas import tpu_sc as plsc

N = 10000
E = 160000
NP = 10240
STRIPE = NP // 16
B = 128
NB = (E // 16) // B
REM = E // 16 - NB * B


def _zero_vmem_rows(ref, nrows, width):
    z = jnp.zeros((16,), jnp.float32)

    def body(i, _):
        for m in range(width // 16):
            ref[i, pl.ds(m * 16, 16)] = z
        return 0

    lax.fori_loop(0, nrows, body, 0)


def _fill_vmem_rows(ref, nrows, width, value):
    v = jnp.full((16,), value, jnp.float32)

    def body(i, _):
        for m in range(width // 16):
            ref[i, pl.ds(m * 16, 16)] = v
        return 0

    lax.fori_loop(0, nrows, body, 0)


def _deg_body(dst, out, zrow, onev, one16, d_v, d16, sem, accum):
    del sem
    c = lax.axis_index("c")
    s = lax.axis_index("s")
    _zero_vmem_rows(zrow, 64, 128)
    _fill_vmem_rows(onev, 128, 128, 1.0)
    _fill_vmem_rows(one16, REM, 128, 1.0)
    for j in range(STRIPE // 64):
        pltpu.sync_copy(zrow, accum.at[pl.ds(s * STRIPE + j * 64, 64), :])
    plsc.subcore_barrier()

    def batch(t, _):
        base = s * (E // 16) + (2 * t + c) * B
        pltpu.sync_copy(dst.at[pl.ds(base, B)], d_v)
        pltpu.sync_copy(onev, accum.at[d_v], add=True)
        return 0

    lax.fori_loop(0, NB // 2, batch, 0)

    @pl.when(c == 0)
    def _():
        rbase = s * (E // 16) + NB * B
        pltpu.sync_copy(dst.at[pl.ds(rbase, REM)], d16)
        pltpu.sync_copy(one16, accum.at[d16], add=True)

    plsc.subcore_barrier()
    pltpu.sync_copy(accum.at[pl.ds(s * STRIPE, STRIPE), :],
                    out.at[c, pl.ds(s * STRIPE, STRIPE), :])


_deg_kernel = pl.kernel(
    _deg_body,
    out_type=jax.ShapeDtypeStruct((2, NP, 128), jnp.float32),
    mesh=plsc.VectorSubcoreMesh(core_axis_name="c", subcore_axis_name="s"),
    scratch_types=[
        pltpu.VMEM((64, 128), jnp.float32),
        pltpu.VMEM((128, 128), jnp.float32),
        pltpu.VMEM((REM, 128), jnp.float32),
        pltpu.VMEM((B,), jnp.int32),
        pltpu.VMEM((REM,), jnp.int32),
        pltpu.SemaphoreType.DMA,
        pltpu.VMEM_SHARED((NP, 128), jnp.float32),
    ],
)


NROW = E // B
NBT = NROW // 16


def _agg_body(nslab, hs, ec, out, zrow, sd0, sd1, sd2, g0, g1, rows0, rows1,
              semi0, semi1, semg0, semg1, sems0, sems1, sem, accum):
    c = lax.axis_index("c")
    s = lax.axis_index("s")
    spc = nslab // 2
    _zero_vmem_rows(zrow, 64, 128)
    sd = (sd0, sd1, sd2)
    g = (g0, g1)
    rows = (rows0, rows1)
    semi = (semi0, semi1)
    semg = (semg0, semg1)
    sems = (sems0, sems1)
    brow = s * NBT + jnp.minimum(s, 2)

    def gcompute(gj, sdj, off):
        for m in range(B // 16):
            gj[pl.ds(m * 16, 16)] = sdj[0, pl.ds(m * 16, 16)] + off

    for k in range(spc):
        slab = c * spc + k
        off = slab * N
        for j in range(STRIPE // 64):
            pltpu.sync_copy(zrow, accum.at[pl.ds(s * STRIPE + j * 64, 64), :])
        plsc.subcore_barrier()

        @pl.when(s < 2)
        def _():
            pltpu.sync_copy(ec.at[brow + NBT], sd0)
            gcompute(g0, sd0, off)
            pltpu.async_copy(hs.at[g0], rows0, sem).wait()
            pltpu.sync_copy(rows0, accum.at[sd0.at[1]], add=True)

        pltpu.sync_copy(ec.at[brow], sd0)
        gcompute(g0, sd0, off)
        pltpu.async_copy(hs.at[g0], rows0, semg0)
        pltpu.async_copy(ec.at[brow + 1], sd1, semi1)

        def six(t, _):
            for j in range(6):
                b = 6 * t + j
                p2 = j % 2
                p3 = j % 3
                pltpu.make_async_copy(hs.at[g[p2]], rows[p2],
                                      semg[p2]).wait()

                @pl.when(b >= 1)
                def _():
                    pltpu.make_async_copy(
                        rows[1 - p2], accum.at[sd[(j - 1) % 3].at[1]],
                        sems[1 - p2]).wait()

                pltpu.async_copy(rows[p2], accum.at[sd[p3].at[1]],
                                 sems[p2], add=True)

                @pl.when(b + 1 < NBT)
                def _():
                    pltpu.make_async_copy(ec.at[brow + b + 1],
                                          sd[(j + 1) % 3],
                                          semi[(j + 1) % 2]).wait()
                    gcompute(g[(j + 1) % 2], sd[(j + 1) % 3], off)
                    pltpu.async_copy(hs.at[g[(j + 1) % 2]],
                                     rows[(j + 1) % 2], semg[(j + 1) % 2])

                @pl.when(b + 2 < NBT)
                def _():
                    pltpu.async_copy(ec.at[brow + b + 2], sd[(j + 2) % 3],
                                     semi[(j + 2) % 2])
            return 0

        lax.fori_loop(0, NBT // 6, six, 0)
        pltpu.make_async_copy(rows[1], accum.at[sd[2].at[1]], sems[1]).wait()
        plsc.subcore_barrier()

        @pl.when(s < 15)
        def _():
            pltpu.sync_copy(
                accum.at[pl.ds(s * STRIPE, STRIPE), :],
                out.at[pl.ds(s * STRIPE, STRIPE), pl.ds(slab * 128, 128)])

        @pl.when(s == 15)
        def _():
            pltpu.sync_copy(
                accum.at[pl.ds(15 * STRIPE, N - 15 * STRIPE), :],
                out.at[pl.ds(15 * STRIPE, N - 15 * STRIPE),
                       pl.ds(slab * 128, 128)])


@functools.cache
def _make_agg(nslab):
    return pl.kernel(
        functools.partial(_agg_body, nslab),
        out_type=jax.ShapeDtypeStruct((N, nslab * 128), jnp.float32),
        mesh=plsc.VectorSubcoreMesh(core_axis_name="c", subcore_axis_name="s"),
        scratch_types=[
            pltpu.VMEM((64, 128), jnp.float32),
            pltpu.VMEM((2, B), jnp.int32),
            pltpu.VMEM((2, B), jnp.int32),
            pltpu.VMEM((2, B), jnp.int32),
            pltpu.VMEM((B,), jnp.int32),
            pltpu.VMEM((B,), jnp.int32),
            pltpu.VMEM((B, 128), jnp.float32),
            pltpu.VMEM((B, 128), jnp.float32),
            pltpu.SemaphoreType.DMA,
            pltpu.SemaphoreType.DMA,
            pltpu.SemaphoreType.DMA,
            pltpu.SemaphoreType.DMA,
            pltpu.SemaphoreType.DMA,
            pltpu.SemaphoreType.DMA,
            pltpu.SemaphoreType.DMA,
            pltpu.VMEM_SHARED((NP, 128), jnp.float32),
        ],
    )


def _dis(degp_ref):
    return lax.rsqrt(degp_ref[0, :, 0:1] + degp_ref[1, :, 0:1] + 1.0)


def _scale_body(feat_ref, degp_ref, xs_n_ref, xs_s_ref):
    xs = feat_ref[...] * _dis(degp_ref)
    xs_n_ref[...] = xs
    xs_s_ref[...] = xs[None]


def _scale(features, degp):
    fin = features.shape[1]
    nslab = fin // 128
    return pl.pallas_call(
        _scale_body,
        grid=(N // 400, nslab),
        in_specs=[
            pl.BlockSpec((400, 128), lambda i, j: (i, j)),
            pl.BlockSpec((2, 400, 128), lambda i, j: (0, i, 0)),
        ],
        out_specs=[
            pl.BlockSpec((400, 128), lambda i, j: (i, j)),
            pl.BlockSpec((1, 400, 128), lambda i, j: (j, i, 0)),
        ],
        out_shape=[
            jax.ShapeDtypeStruct((N, fin), jnp.float32),
            jax.ShapeDtypeStruct((nslab, N, 128), jnp.float32),
        ],
    )(features, degp)


def _mm2_body(z_ref, xs_ref, b1_ref, w1_ref, w2_ref, degp_ref,
              hs_n_ref, hs_s_ref):
    d = _dis(degp_ref)
    y = d * (z_ref[...] + xs_ref[...])
    x2 = jnp.maximum(
        jnp.dot(y, w1_ref[...], preferred_element_type=jnp.float32)
        + b1_ref[...], 0.0)
    h = jnp.dot(x2, w2_ref[...], preferred_element_type=jnp.float32)
    hs_n_ref[...] = h * d
    hs_s_ref[...] = (h * d)[None]


def _mm2(z1, xs1, b1, w1, w2, degp):
    fin = w1.shape[0]
    fmid = w1.shape[1]
    fout = w2.shape[1]
    nslab = fout // 128
    return pl.pallas_call(
        _mm2_body,
        grid=(N // 400, nslab),
        in_specs=[
            pl.BlockSpec((400, fin), lambda i, j: (i, 0)),
            pl.BlockSpec((400, fin), lambda i, j: (i, 0)),
            pl.BlockSpec((1, fmid), lambda i, j: (0, 0)),
            pl.BlockSpec((fin, fmid), lambda i, j: (0, 0)),
            pl.BlockSpec((fmid, 128), lambda i, j: (0, j)),
            pl.BlockSpec((2, 400, 128), lambda i, j: (0, i, 0)),
        ],
        out_specs=[
            pl.BlockSpec((400, 128), lambda i, j: (i, j)),
            pl.BlockSpec((1, 400, 128), lambda i, j: (j, i, 0)),
        ],
        out_shape=[
            jax.ShapeDtypeStruct((N, fout), jnp.float32),
            jax.ShapeDtypeStruct((nslab, N, 128), jnp.float32),
        ],
    )(z1, xs1, b1.reshape(1, fmid), w1, w2, degp)


def _mmf_body(agg_ref, hsp_ref, bp_ref, w_ref, degp_ref, hs_n_ref, hs_s_ref):
    d = _dis(degp_ref)
    x = jnp.maximum(d * (agg_ref[...] + hsp_ref[...]) + bp_ref[...], 0.0)
    h = jnp.dot(x, w_ref[...], preferred_element_type=jnp.float32)
    hs_n_ref[...] = h * d
    hs_s_ref[...] = (h * d)[None]


def _mm_fused(agg, hs_p, b_p, w, degp):
    fin, fout = w.shape
    nslab = fout // 128
    return pl.pallas_call(
        _mmf_body,
        grid=(N // 400, nslab),
        in_specs=[
            pl.BlockSpec((400, fin), lambda i, j: (i, 0)),
            pl.BlockSpec((400, fin), lambda i, j: (i, 0)),
            pl.BlockSpec((1, fin), lambda i, j: (0, 0)),
            pl.BlockSpec((fin, 128), lambda i, j: (0, j)),
            pl.BlockSpec((2, 400, 128), lambda i, j: (0, i, 0)),
        ],
        out_specs=[
            pl.BlockSpec((400, 128), lambda i, j: (i, j)),
            pl.BlockSpec((1, 400, 128), lambda i, j: (j, i, 0)),
        ],
        out_shape=[
            jax.ShapeDtypeStruct((N, fout), jnp.float32),
            jax.ShapeDtypeStruct((nslab, N, 128), jnp.float32),
        ],
    )(agg, hs_p, b_p.reshape(1, fin), w, degp)


def _elem_body(agg_ref, hs_ref, degp_ref, b_ref, out_ref):
    out_ref[...] = (_dis(degp_ref) * (agg_ref[...] + hs_ref[...])
                    + b_ref[...])


def _elem(agg, hs_n, degp, bias):
    f = agg.shape[1]
    return pl.pallas_call(
        _elem_body,
        grid=(N // 400, f // 128),
        in_specs=[
            pl.BlockSpec((400, 128), lambda i, j: (i, j)),
            pl.BlockSpec((400, 128), lambda i, j: (i, j)),
            pl.BlockSpec((2, 400, 128), lambda i, j: (0, i, 0)),
            pl.BlockSpec((1, 128), lambda i, j: (0, j)),
        ],
        out_specs=pl.BlockSpec((400, 128), lambda i, j: (i, j)),
        out_shape=jax.ShapeDtypeStruct((N, f), jnp.float32),
    )(agg, hs_n, degp, bias.reshape(1, f))


def _agg_of(hs_s, ec):
    nslab = hs_s.shape[0]
    return _make_agg(nslab)(hs_s.reshape(nslab * N, 128), ec)


def kernel(edge_indices, features, W1, b1, W2, b2, W3, b3):
    edge = jnp.asarray(edge_indices, jnp.int32)
    dst = edge[1]
    ec = jnp.transpose(edge.reshape(2, NROW, B), (1, 0, 2))
    degp = _deg_kernel(dst)
    xs1_n, xs1_s = _scale(features, degp)
    z1 = _agg_of(xs1_s, ec)
    hs_n2, hs_s2 = _mm2(z1, xs1_n, b1, W1, W2, degp)
    agg2 = _agg_of(hs_s2, ec)
    hs_n3, hs_s3 = _mm_fused(agg2, hs_n2, b2, W3, degp)
    agg3 = _agg_of(hs_s3, ec)
    return _elem(agg3, hs_n3, degp, b3)
